# Initial kernel scaffold; baseline (speedup 1.0000x reference)
import jax
import jax.numpy as jnp
from jax.experimental import pallas as pl


def kernel(input_ids, table):
    raise NotImplementedError("write your pallas kernel here")


# speedup vs baseline: 7.3399x; 7.3399x over previous
import functools

import jax
import jax.numpy as jnp
from jax import lax
from jax.experimental import pallas as pl
from jax.experimental.pallas import tpu as pltpu, tpu_sc as plsc

VOCAB = 100000
EMBED = 128
TOTAL = 1024 * 200

NC = 2
NS = 16
NW = NC * NS
PER_W = TOTAL // NW
G = 100
K = 4
CHUNK = G * K
NGROUP = PER_W // G
NCHUNK = PER_W // CHUNK

_mesh = plsc.VectorSubcoreMesh(core_axis_name="c", subcore_axis_name="s")


@functools.partial(
    pl.kernel,
    out_type=jax.ShapeDtypeStruct((TOTAL, EMBED), jnp.float32),
    mesh=_mesh,
    scratch_types=[
        pltpu.VMEM((NGROUP, G), jnp.int32),
        pltpu.VMEM((CHUNK, EMBED), jnp.float32),
        pltpu.SemaphoreType.DMA,
        pltpu.SemaphoreType.DMA,
    ],
)
def _emb_lookup(ids_hbm, table_hbm, out_hbm, idx_v, rows, gsem, osem):
    wid = lax.axis_index("s") * NC + lax.axis_index("c")
    pltpu.sync_copy(ids_hbm.at[wid], idx_v)
    base = wid * PER_W

    def chunk_body(c, carry):
        for j in range(K):
            pltpu.async_copy(
                table_hbm.at[idx_v.at[c * K + j]],
                rows.at[pl.ds(j * G, G)],
                gsem,
            )
        for j in range(K):
            pltpu.make_async_copy(
                table_hbm.at[idx_v.at[c * K + j]],
                rows.at[pl.ds(j * G, G)],
                gsem,
            ).wait()
        pltpu.async_copy(rows, out_hbm.at[pl.ds(base + c * CHUNK, CHUNK)], osem)
        pltpu.make_async_copy(
            rows, out_hbm.at[pl.ds(base + c * CHUNK, CHUNK)], osem
        ).wait()
        return carry

    lax.fori_loop(0, NCHUNK, chunk_body, 0)


def kernel(input_ids, table):
    b, s = input_ids.shape
    ids = input_ids.reshape(NW, NGROUP, G).astype(jnp.int32)
    out = _emb_lookup(ids, table)
    return out.reshape(b, s, EMBED)


# speedup vs baseline: 8.1125x; 1.1053x over previous
import functools

import jax
import jax.numpy as jnp
from jax import lax
from jax.experimental import pallas as pl
from jax.experimental.pallas import tpu as pltpu, tpu_sc as plsc

VOCAB = 100000
EMBED = 128
TOTAL = 1024 * 200

NC = 2
NS = 16
NW = NC * NS
PER_W = TOTAL // NW
G = 100
K = 4
CHUNK = G * K
NGROUP = PER_W // G
NCHUNK = PER_W // CHUNK

_mesh = plsc.VectorSubcoreMesh(core_axis_name="c", subcore_axis_name="s")


@functools.partial(
    pl.kernel,
    out_type=jax.ShapeDtypeStruct((TOTAL, EMBED), jnp.float32),
    mesh=_mesh,
    scratch_types=[
        pltpu.VMEM((NGROUP, G), jnp.int32),
        pltpu.VMEM((CHUNK, EMBED), jnp.float32),
        pltpu.VMEM((CHUNK, EMBED), jnp.float32),
        pltpu.SemaphoreType.DMA,
        pltpu.SemaphoreType.DMA,
        pltpu.SemaphoreType.DMA,
        pltpu.SemaphoreType.DMA,
    ],
)
def _emb_lookup(ids_hbm, table_hbm, out_hbm, idx_v, rows0, rows1, g0, g1, o0, o1):
    wid = lax.axis_index("s") * NC + lax.axis_index("c")
    pltpu.sync_copy(ids_hbm.at[wid], idx_v)
    base = wid * PER_W

    def fire_gather(c, rows, gsem):
        for j in range(K):
            pltpu.async_copy(
                table_hbm.at[idx_v.at[c * K + j]],
                rows.at[pl.ds(j * G, G)],
                gsem,
            )

    def wait_gather(rows, gsem):
        for j in range(K):
            pltpu.make_async_copy(
                table_hbm.at[idx_v.at[j]], rows.at[pl.ds(j * G, G)], gsem
            ).wait()

    def fire_out(c, rows, osem):
        pltpu.async_copy(rows, out_hbm.at[pl.ds(base + c * CHUNK, CHUNK)], osem)

    def wait_out(rows, osem):
        pltpu.make_async_copy(
            rows, out_hbm.at[pl.ds(base, CHUNK)], osem
        ).wait()

    fire_gather(0, rows0, g0)
    fire_gather(1, rows1, g1)
    wait_gather(rows0, g0)
    fire_out(0, rows0, o0)

    def pair_body(p, carry):
        ca = 2 * p + 2
        wait_out(rows0, o0)
        fire_gather(ca, rows0, g0)
        wait_gather(rows1, g1)
        fire_out(ca - 1, rows1, o1)
        wait_out(rows1, o1)
        fire_gather(ca + 1, rows1, g1)
        wait_gather(rows0, g0)
        fire_out(ca, rows0, o0)
        return carry

    lax.fori_loop(0, (NCHUNK - 2) // 2, pair_body, 0)

    wait_gather(rows1, g1)
    fire_out(NCHUNK - 1, rows1, o1)
    wait_out(rows0, o0)
    wait_out(rows1, o1)


def kernel(input_ids, table):
    b, s = input_ids.shape
    ids = input_ids.reshape(NW, NGROUP, G).astype(jnp.int32)
    out = _emb_lookup(ids, table)
    return out.reshape(b, s, EMBED)
